# baseline (speedup 1.0000x reference)
import functools

import jax
import jax.numpy as jnp
from jax import lax
from jax.experimental import pallas as pl
from jax.experimental.pallas import tpu as pltpu
from jax.experimental.pallas import tpu_sc as plsc

N_FIELDS = 26
VOCAB = 100000
EMBED_DIM = 32
DENSE_DIM = 13
BATCH = 4096
NV = N_FIELDS + 1
_B_PER_W = 128
_CHUNK = EMBED_DIM * _B_PER_W
_L = 16


def _make_sc_gather(nf):
    info = plsc.get_sparse_core_info()
    NC, NS = info.num_cores, info.num_subcores
    NW = NC * NS
    assert BATCH == NW * _B_PER_W
    mesh = plsc.VectorSubcoreMesh(core_axis_name="c", subcore_axis_name="s")
    FD = nf * EMBED_DIM

    @functools.partial(
        pl.kernel,
        mesh=mesh,
        out_type=jax.ShapeDtypeStruct((NW, FD * _B_PER_W), jnp.float32),
        scratch_types=[
            pltpu.VMEM((_B_PER_W,), jnp.int32),
            pltpu.VMEM((2, _CHUNK), jnp.int32),
            pltpu.VMEM((FD * _B_PER_W,), jnp.float32),
            pltpu.SemaphoreType.DMA,
        ],
        compiler_params=pltpu.CompilerParams(use_tc_tiling_on_sc=False),
    )
    def gather_k(tab_hbm, vt_hbm, out_hbm, vv, idx_v, strip_v, sem):
        wid = lax.axis_index("s") * NC + lax.axis_index("c")
        b0 = wid * _B_PER_W
        for f in range(nf):
            pltpu.sync_copy(vt_hbm.at[f, pl.ds(b0, _B_PER_W)], vv)
            buf = f % 2
            for l8 in range(_B_PER_W // _L):
                v16 = vv[pl.ds(l8 * _L, _L)]
                for d in range(EMBED_DIM):
                    idx_v[buf, pl.ds(d * _B_PER_W + l8 * _L, _L)] = (
                        v16 + (f * EMBED_DIM + d) * VOCAB
                    )
            pltpu.async_copy(
                tab_hbm.at[idx_v.at[buf]],
                strip_v.at[pl.ds(f * _CHUNK, _CHUNK)],
                sem,
            )
        pltpu.make_async_copy(
            tab_hbm.at[pl.ds(0, FD * _B_PER_W)], strip_v, sem
        ).wait()
        pltpu.sync_copy(strip_v, out_hbm.at[wid])

    return gather_k


_FIELD_GROUPS = (7, 7, 6, 6)
_sc_gathers = {nf: _make_sc_gather(nf) for nf in set(_FIELD_GROUPS)}


def _dense_body(xt_ref, embt_ref, bW0t, bb0c, bW1t, bb1c, bW2t, bb2c,
                tW0t, tb0c, tW1t, tb1c, tW2t, tb2c, out_ref):
    x = xt_ref[...]
    h = jnp.maximum(jnp.dot(bW0t[...], x, preferred_element_type=jnp.float32) + bb0c[...], 0.0)
    h = jnp.maximum(jnp.dot(bW1t[...], h, preferred_element_type=jnp.float32) + bb1c[...], 0.0)
    h = jnp.maximum(jnp.dot(bW2t[...], h, preferred_element_type=jnp.float32) + bb2c[...], 0.0)
    ft = jnp.concatenate([h, embt_ref[...]], axis=0)
    f3 = ft.reshape(NV, EMBED_DIM, ft.shape[1])
    parts = []
    for i in range(1, NV):
        parts.append(jnp.sum(f3[:i] * f3[i][None], axis=1))
    inter_t = jnp.concatenate(parts, axis=0)
    top_t = jnp.concatenate([h, inter_t], axis=0)
    t = jnp.maximum(jnp.dot(tW0t[...], top_t, preferred_element_type=jnp.float32) + tb0c[...], 0.0)
    t = jnp.maximum(jnp.dot(tW1t[...], t, preferred_element_type=jnp.float32) + tb1c[...], 0.0)
    o = jnp.dot(tW2t[...], t, preferred_element_type=jnp.float32) + tb2c[...]
    out_ref[...] = 1.0 / (1.0 + jnp.exp(-o))


def _dense_call(xt, embt, *ws):
    Bblk = 512
    grid = (BATCH // Bblk,)
    full = lambda a: pl.BlockSpec(a.shape, lambda i: (0,) * a.ndim)
    out = pl.pallas_call(
        _dense_body,
        grid=grid,
        in_specs=[
            pl.BlockSpec((DENSE_DIM, Bblk), lambda i: (0, i)),
            pl.BlockSpec((N_FIELDS * EMBED_DIM, Bblk), lambda i: (0, i)),
        ] + [full(w) for w in ws],
        out_specs=pl.BlockSpec((1, Bblk), lambda i: (0, i)),
        out_shape=jax.ShapeDtypeStruct((1, BATCH), jnp.float32),
    )(xt, embt, *ws)
    return out.reshape(BATCH, 1)


def kernel(dense_x, sparse_indices, tables, bW0, bb0, bW1, bb1, bW2, bb2,
           tW0, tb0, tW1, tb1, tW2, tb2):
    tabt3 = jnp.transpose(tables, (0, 2, 1))
    vt = jnp.transpose(sparse_indices.astype(jnp.int32))
    raws = []
    f0 = 0
    for nf in _FIELD_GROUPS:
        tab1d_g = tabt3[f0:f0 + nf].reshape(-1)
        raws.append(_sc_gathers[nf](tab1d_g, vt[f0:f0 + nf]))
        f0 += nf
    raw = jnp.concatenate(raws, axis=1)
    embt = (
        raw.reshape(32, N_FIELDS * EMBED_DIM, _B_PER_W)
        .transpose(1, 0, 2)
        .reshape(N_FIELDS * EMBED_DIM, BATCH)
    )
    return _dense_call(
        dense_x.T, embt,
        bW0.T, bb0.reshape(-1, 1), bW1.T, bb1.reshape(-1, 1), bW2.T, bb2.reshape(-1, 1),
        tW0.T, tb0.reshape(-1, 1), tW1.T, tb1.reshape(-1, 1), tW2.T, tb2.reshape(-1, 1),
    )


# speedup vs baseline: 1.2114x; 1.2114x over previous
import functools

import jax
import jax.numpy as jnp
from jax import lax
from jax.experimental import pallas as pl
from jax.experimental.pallas import tpu as pltpu
from jax.experimental.pallas import tpu_sc as plsc

N_FIELDS = 26
VOCAB = 100000
EMBED_DIM = 32
DENSE_DIM = 13
BATCH = 4096
NV = N_FIELDS + 1
_B_PER_W = 128
_CHUNK = EMBED_DIM * _B_PER_W
_L = 16


def _make_sc_gather(nf):
    info = plsc.get_sparse_core_info()
    NC, NS = info.num_cores, info.num_subcores
    NW = NC * NS
    assert BATCH == NW * _B_PER_W
    mesh = plsc.VectorSubcoreMesh(core_axis_name="c", subcore_axis_name="s")
    FD = nf * EMBED_DIM

    @functools.partial(
        pl.kernel,
        mesh=mesh,
        out_type=jax.ShapeDtypeStruct((NW, FD * _B_PER_W), jnp.float32),
        scratch_types=[
            pltpu.VMEM((_B_PER_W,), jnp.int32),
            pltpu.VMEM((2, _CHUNK), jnp.int32),
            pltpu.VMEM((FD * _B_PER_W,), jnp.float32),
            pltpu.SemaphoreType.DMA,
        ],
        compiler_params=pltpu.CompilerParams(use_tc_tiling_on_sc=False),
    )
    def gather_k(tab_hbm, vt_hbm, out_hbm, vv, idx_v, strip_v, sem):
        wid = lax.axis_index("s") * NC + lax.axis_index("c")
        b0 = wid * _B_PER_W
        for f in range(nf):
            pltpu.sync_copy(vt_hbm.at[f, pl.ds(b0, _B_PER_W)], vv)
            buf = f % 2
            for l8 in range(_B_PER_W // _L):
                v16 = vv[pl.ds(l8 * _L, _L)]
                for d in range(EMBED_DIM):
                    idx_v[buf, pl.ds(d * _B_PER_W + l8 * _L, _L)] = (
                        v16 + (f * EMBED_DIM + d) * VOCAB
                    )
            pltpu.async_copy(
                tab_hbm.at[idx_v.at[buf]],
                strip_v.at[pl.ds(f * _CHUNK, _CHUNK)],
                sem,
            )
        pltpu.make_async_copy(
            tab_hbm.at[pl.ds(0, FD * _B_PER_W)], strip_v, sem
        ).wait()
        pltpu.sync_copy(strip_v, out_hbm.at[wid])

    return gather_k


_sc_gather = _make_sc_gather(N_FIELDS)


def _dense_body(xt_ref, embt_ref, bW0t, bb0c, bW1t, bb1c, bW2t, bb2c,
                tW0t, tb0c, tW1t, tb1c, tW2t, tb2c, out_ref):
    x = xt_ref[...]
    h = jnp.maximum(jnp.dot(bW0t[...], x, preferred_element_type=jnp.float32) + bb0c[...], 0.0)
    h = jnp.maximum(jnp.dot(bW1t[...], h, preferred_element_type=jnp.float32) + bb1c[...], 0.0)
    h = jnp.maximum(jnp.dot(bW2t[...], h, preferred_element_type=jnp.float32) + bb2c[...], 0.0)
    ft = jnp.concatenate([h, embt_ref[...]], axis=0)
    f3 = ft.reshape(NV, EMBED_DIM, ft.shape[1])
    parts = []
    for i in range(1, NV):
        parts.append(jnp.sum(f3[:i] * f3[i][None], axis=1))
    inter_t = jnp.concatenate(parts, axis=0)
    top_t = jnp.concatenate([h, inter_t], axis=0)
    t = jnp.maximum(jnp.dot(tW0t[...], top_t, preferred_element_type=jnp.float32) + tb0c[...], 0.0)
    t = jnp.maximum(jnp.dot(tW1t[...], t, preferred_element_type=jnp.float32) + tb1c[...], 0.0)
    o = jnp.dot(tW2t[...], t, preferred_element_type=jnp.float32) + tb2c[...]
    out_ref[...] = 1.0 / (1.0 + jnp.exp(-o))


def _dense_call(xt, embt, *ws):
    Bblk = 1024
    grid = (BATCH // Bblk,)
    full = lambda a: pl.BlockSpec(a.shape, lambda i: (0,) * a.ndim)
    out = pl.pallas_call(
        _dense_body,
        grid=grid,
        in_specs=[
            pl.BlockSpec((DENSE_DIM, Bblk), lambda i: (0, i)),
            pl.BlockSpec((N_FIELDS * EMBED_DIM, Bblk), lambda i: (0, i)),
        ] + [full(w) for w in ws],
        out_specs=pl.BlockSpec((1, Bblk), lambda i: (0, i)),
        out_shape=jax.ShapeDtypeStruct((1, BATCH), jnp.float32),
    )(xt, embt, *ws)
    return out.reshape(BATCH, 1)


def kernel(dense_x, sparse_indices, tables, bW0, bb0, bW1, bb1, bW2, bb2,
           tW0, tb0, tW1, tb1, tW2, tb2):
    tab1d = jnp.transpose(tables, (0, 2, 1)).reshape(-1)
    vt = jnp.transpose(sparse_indices.astype(jnp.int32))
    raw = _sc_gather(tab1d, vt)
    embt = (
        raw.reshape(32, N_FIELDS * EMBED_DIM, _B_PER_W)
        .transpose(1, 0, 2)
        .reshape(N_FIELDS * EMBED_DIM, BATCH)
    )
    return _dense_call(
        dense_x.T, embt,
        bW0.T, bb0.reshape(-1, 1), bW1.T, bb1.reshape(-1, 1), bW2.T, bb2.reshape(-1, 1),
        tW0.T, tb0.reshape(-1, 1), tW1.T, tb1.reshape(-1, 1), tW2.T, tb2.reshape(-1, 1),
    )
